# baseline (speedup 1.0000x reference)
import jax
import jax.numpy as jnp
from jax.experimental import pallas as pl
from jax.experimental.pallas import tpu as pltpu

_NUM_TREES = 16
_NUM_NODES = 15
_BLOCK_N = 2000


def _forest_max(a_ref):
    m = None
    for t in range(_NUM_TREES):
        base = t * _NUM_NODES
        v = [None] * _NUM_NODES
        for i in range(_NUM_NODES - 1, -1, -1):
            ai = a_ref[base + i : base + i + 1, :]
            if 2 * i + 1 >= _NUM_NODES:
                v[i] = ai
            else:
                v[i] = jnp.maximum(v[2 * i + 1], v[2 * i + 2]) + ai
        m = v[0] if m is None else jnp.maximum(m, v[0])
    return m


def _fis_kernel(a_ref, x_ref, o_ref, em_ref):
    @pl.when(pl.program_id(0) == 0)
    def _():
        em_ref[...] = jnp.exp(_forest_max(a_ref))

    em = em_ref[...]
    o_ref[...] = jnp.maximum(x_ref[...], 0.0) * em + (em - 1.0)


def kernel(x, alphas):
    n, c = x.shape
    a2d = alphas.reshape(_NUM_TREES * _NUM_NODES, c)

    return pl.pallas_call(
        _fis_kernel,
        grid=(pl.cdiv(n, _BLOCK_N),),
        in_specs=[
            pl.BlockSpec(memory_space=pltpu.VMEM),
            pl.BlockSpec((_BLOCK_N, c), lambda i: (i, 0)),
        ],
        out_specs=pl.BlockSpec((_BLOCK_N, c), lambda i: (i, 0)),
        out_shape=jax.ShapeDtypeStruct((n, c), x.dtype),
        scratch_shapes=[pltpu.VMEM((1, c), x.dtype)],
        compiler_params=pltpu.CompilerParams(
            dimension_semantics=("arbitrary",),
        ),
    )(a2d, x)


# speedup vs baseline: 1.1322x; 1.1322x over previous
import jax
import jax.numpy as jnp
from jax.experimental import pallas as pl
from jax.experimental.pallas import tpu as pltpu

_NUM_TREES = 16
_NUM_NODES = 15
_BLOCK_N = 5000


def _forest_max(a_ref):
    m = None
    for t in range(_NUM_TREES):
        base = t * _NUM_NODES
        v = [None] * _NUM_NODES
        for i in range(_NUM_NODES - 1, -1, -1):
            ai = a_ref[base + i : base + i + 1, :]
            if 2 * i + 1 >= _NUM_NODES:
                v[i] = ai
            else:
                v[i] = jnp.maximum(v[2 * i + 1], v[2 * i + 2]) + ai
        m = v[0] if m is None else jnp.maximum(m, v[0])
    return m


def _fis_kernel(a_ref, x_ref, o_ref, em_ref):
    @pl.when(pl.program_id(0) == 0)
    def _():
        em_ref[...] = jnp.exp(_forest_max(a_ref))

    em = em_ref[...]
    o_ref[...] = jnp.maximum(x_ref[...], 0.0) * em + (em - 1.0)


def kernel(x, alphas):
    n, c = x.shape
    a2d = alphas.reshape(_NUM_TREES * _NUM_NODES, c)

    return pl.pallas_call(
        _fis_kernel,
        grid=(pl.cdiv(n, _BLOCK_N),),
        in_specs=[
            pl.BlockSpec(memory_space=pltpu.VMEM),
            pl.BlockSpec((_BLOCK_N, c), lambda i: (i, 0)),
        ],
        out_specs=pl.BlockSpec((_BLOCK_N, c), lambda i: (i, 0)),
        out_shape=jax.ShapeDtypeStruct((n, c), x.dtype),
        scratch_shapes=[pltpu.VMEM((1, c), x.dtype)],
        compiler_params=pltpu.CompilerParams(
            dimension_semantics=("arbitrary",),
        ),
    )(a2d, x)
